# Initial kernel scaffold; baseline (speedup 1.0000x reference)
import jax
import jax.numpy as jnp
from jax.experimental import pallas as pl


def kernel(x, mean_list, scale_list, weight_logits):
    raise NotImplementedError("write your pallas kernel here")


# speedup vs baseline: 8.0480x; 8.0480x over previous
import functools

import jax
import jax.numpy as jnp
import numpy as np
from jax.experimental import pallas as pl
from jax.experimental.pallas import tpu as pltpu

_K = 8
_BR = 256


def _body(mean_ref, ninv_ref, cns_ref, x_ref, o_ref):
    x = x_ref[...]
    lmax = None
    for k in range(_K):
        d = x - mean_ref[k]
        l = d * d * ninv_ref[k] + cns_ref[k]
        lmax = l if lmax is None else jnp.maximum(lmax, l)
    acc = None
    for k in range(_K):
        d = x - mean_ref[k]
        l = d * d * ninv_ref[k] + cns_ref[k]
        e = jnp.exp(l - lmax)
        acc = e if acc is None else acc + e
    o_ref[...] = lmax + jnp.log(acc)


@jax.jit
def kernel(x, mean_list, scale_list, weight_logits):
    B, Z = x.shape
    K = mean_list.shape[-1]
    scale = jax.nn.softplus(scale_list)
    ninv = -0.5 / (scale * scale)
    log_w = jax.nn.log_softmax(weight_logits, axis=-1)
    cns = -jnp.log(scale) - 0.5 * np.log(2.0 * np.pi) + log_w[:, None, :]
    mean_t = jnp.transpose(mean_list, (2, 0, 1))
    ninv_t = jnp.transpose(ninv, (2, 0, 1))
    cns_t = jnp.transpose(cns, (2, 0, 1))

    grid = (B // _BR,)
    coeff_spec = pl.BlockSpec((K, 1, Z), lambda i: (0, 0, 0))
    return pl.pallas_call(
        _body,
        grid=grid,
        in_specs=[
            coeff_spec,
            coeff_spec,
            coeff_spec,
            pl.BlockSpec((_BR, Z), lambda i: (i, 0)),
        ],
        out_specs=pl.BlockSpec((_BR, Z), lambda i: (i, 0)),
        out_shape=jax.ShapeDtypeStruct((B, Z), x.dtype),
    )(mean_t, ninv_t, cns_t, x)
